# Initial kernel scaffold; baseline (speedup 1.0000x reference)
import jax
import jax.numpy as jnp
from jax.experimental import pallas as pl


def kernel(x, edge_index, W, att):
    raise NotImplementedError("write your pallas kernel here")


# speedup vs baseline: 1.3037x; 1.3037x over previous
import functools

import jax
import jax.numpy as jnp
from jax import lax
from jax.experimental import pallas as pl
from jax.experimental.pallas import tpu as pltpu
from jax.experimental.pallas import tpu_sc as plsc

N = 10000
E = 160000
D_IN = 256
D_OUT = 256
HEADS = 8
ALPHA = 0.2

NC = 2
NS = 16
NW = NC * NS
EPW = E // NW
CHUNK = 128
NFULL = EPW // CHUNK
TAIL = EPW - NFULL * CHUNK

BN = 1000
TW = 128


def _proj_body(x_ref, wc_ref, a12_ref, h_ref, ss_ref):
    h = jnp.dot(x_ref[...], wc_ref[...], preferred_element_type=jnp.float32)
    h_ref[...] = h
    ss_ref[...] = jnp.dot(h, a12_ref[...], preferred_element_type=jnp.float32)


def _project(x, wc, a12):
    return pl.pallas_call(
        _proj_body,
        grid=(N // BN,),
        in_specs=[
            pl.BlockSpec((BN, D_IN), lambda i: (i, 0)),
            pl.BlockSpec((D_IN, HEADS * D_OUT), lambda i: (0, 0)),
            pl.BlockSpec((HEADS * D_OUT, 2 * TW), lambda i: (0, 0)),
        ],
        out_specs=[
            pl.BlockSpec((BN, HEADS * D_OUT), lambda i: (i, 0)),
            pl.BlockSpec((BN, 2 * TW), lambda i: (i, 0)),
        ],
        out_shape=[
            jax.ShapeDtypeStruct((N, HEADS * D_OUT), jnp.float32),
            jax.ShapeDtypeStruct((N, 2 * TW), jnp.float32),
        ],
    )(x, wc, a12)


def _edge_kernel(ta_hbm, tb_hbm, src_hbm, dst_hbm, out_hbm,
                 is_v, id_v, ra_v, rb_v, ro_v,
                 is_t, id_t, ra_t, rb_t, ro_t, sem):
    wid = lax.axis_index("s") * NC + lax.axis_index("c")
    base = wid * EPW

    def compute(ra, rb, ro, size):
        def body(e, carry):
            logit = ra[e, pl.ds(0, NS)] + rb[e, pl.ds(0, NS)]
            lrelu = jnp.where(logit > 0, logit, ALPHA * logit)
            ro[e] = jnp.exp(-lrelu)
            return carry
        lax.fori_loop(0, size, body, 0)

    def do_chunk(cbase, is_b, id_b, ra_b, rb_b, ro_b, size):
        pltpu.sync_copy(src_hbm.at[pl.ds(cbase, size)], is_b)
        pltpu.sync_copy(dst_hbm.at[pl.ds(cbase, size)], id_b)
        pltpu.async_copy(ta_hbm.at[is_b], ra_b, sem).wait()
        pltpu.async_copy(tb_hbm.at[id_b], rb_b, sem).wait()
        compute(ra_b, rb_b, ro_b, size)
        pltpu.sync_copy(ro_b, out_hbm.at[pl.ds(cbase, size)])

    def chunk_loop(i, carry):
        do_chunk(base + i * CHUNK, is_v, id_v, ra_v, rb_v, ro_v, CHUNK)
        return carry
    lax.fori_loop(0, NFULL, chunk_loop, 0)
    do_chunk(base + NFULL * CHUNK, is_t, id_t, ra_t, rb_t, ro_t, TAIL)


def _edge_coeffs(ta, tb, src, dst):
    mesh = plsc.VectorSubcoreMesh(core_axis_name="c", subcore_axis_name="s")
    kern = functools.partial(
        pl.kernel,
        mesh=mesh,
        out_type=jax.ShapeDtypeStruct((E, NS), jnp.float32),
        scratch_types=[
            pltpu.VMEM((CHUNK,), jnp.int32),
            pltpu.VMEM((CHUNK,), jnp.int32),
            pltpu.VMEM((CHUNK, TW), jnp.float32),
            pltpu.VMEM((CHUNK, TW), jnp.float32),
            pltpu.VMEM((CHUNK, NS), jnp.float32),
            pltpu.VMEM((TAIL,), jnp.int32),
            pltpu.VMEM((TAIL,), jnp.int32),
            pltpu.VMEM((TAIL, TW), jnp.float32),
            pltpu.VMEM((TAIL, TW), jnp.float32),
            pltpu.VMEM((TAIL, NS), jnp.float32),
            pltpu.SemaphoreType.DMA,
        ],
    )(_edge_kernel)
    return kern(ta, tb, src, dst)


def _norm_body(hp_ref, rs_ref, o_ref):
    hp = hp_ref[...].reshape(BN, HEADS, D_OUT)
    rs = rs_ref[...][:, :, None]
    o_ref[...] = (hp / rs).reshape(BN, HEADS * D_OUT)


def _normalize(h_prime, rowsum):
    return pl.pallas_call(
        _norm_body,
        grid=(N // BN,),
        in_specs=[
            pl.BlockSpec((BN, HEADS * D_OUT), lambda i: (i, 0)),
            pl.BlockSpec((BN, HEADS), lambda i: (i, 0)),
        ],
        out_specs=pl.BlockSpec((BN, HEADS * D_OUT), lambda i: (i, 0)),
        out_shape=jax.ShapeDtypeStruct((N, HEADS * D_OUT), jnp.float32),
    )(h_prime, rowsum)


@jax.jit
def kernel(x, edge_index, W, att):
    src = edge_index[0, :]
    dst = edge_index[1, :]

    wc = jnp.transpose(W, (1, 0, 2)).reshape(D_IN, HEADS * D_OUT)
    a1 = jnp.zeros((HEADS * D_OUT, TW), jnp.float32)
    a2 = jnp.zeros((HEADS * D_OUT, TW), jnp.float32)
    for h in range(HEADS):
        a1 = a1.at[h * D_OUT:(h + 1) * D_OUT, h].set(att[h, :D_OUT])
        a2 = a2.at[h * D_OUT:(h + 1) * D_OUT, h].set(att[h, D_OUT:])
    a12 = jnp.concatenate([a1, a2], axis=1)

    h_all, ss = _project(x, wc, a12)
    ta = ss[:, :TW]
    tb = ss[:, TW:]

    ee16 = _edge_coeffs(ta, tb, src, dst)
    ee = ee16[:, :HEADS]

    e_rowsum = jax.ops.segment_sum(ee, src, num_segments=N)
    hg = h_all[dst, :].reshape(E, HEADS, D_OUT)
    weighted = (ee[:, :, None] * hg).reshape(E, HEADS * D_OUT)
    h_prime = jax.ops.segment_sum(weighted, src, num_segments=N)

    return _normalize(h_prime, e_rowsum)


# speedup vs baseline: 1.6417x; 1.2593x over previous
import functools

import jax
import jax.numpy as jnp
from jax import lax
from jax.experimental import pallas as pl
from jax.experimental.pallas import tpu as pltpu
from jax.experimental.pallas import tpu_sc as plsc

N = 10000
E = 160000
D_IN = 256
D_OUT = 256
HEADS = 8
ALPHA = 0.2

NC = 2
NS = 16
NW = NC * NS
EPW = E // NW
CHUNK = 128
NFULL = EPW // CHUNK
TAIL = EPW - NFULL * CHUNK

BN = 1000
TW = 128


def _proj_body(x_ref, v12_ref, ss_ref):
    ss_ref[...] = jnp.dot(x_ref[...], v12_ref[...],
                          preferred_element_type=jnp.float32)


def _project(x, v12):
    return pl.pallas_call(
        _proj_body,
        grid=(N // BN,),
        in_specs=[
            pl.BlockSpec((BN, D_IN), lambda i: (i, 0)),
            pl.BlockSpec((D_IN, 2 * TW), lambda i: (0, 0)),
        ],
        out_specs=pl.BlockSpec((BN, 2 * TW), lambda i: (i, 0)),
        out_shape=jax.ShapeDtypeStruct((N, 2 * TW), jnp.float32),
    )(x, v12)


def _edge_kernel(ta_hbm, tb_hbm, src_hbm, dst_hbm, out_hbm,
                 is_v, id_v, ra_v, rb_v, ro_v,
                 is_t, id_t, ra_t, rb_t, ro_t, sem):
    wid = lax.axis_index("s") * NC + lax.axis_index("c")
    base = wid * EPW

    def compute(ra, rb, ro, size):
        def body(e, carry):
            logit = ra[e, pl.ds(0, NS)] + rb[e, pl.ds(0, NS)]
            lrelu = jnp.where(logit > 0, logit, ALPHA * logit)
            ro[e] = jnp.exp(-lrelu)
            return carry
        lax.fori_loop(0, size, body, 0)

    def do_chunk(cbase, is_b, id_b, ra_b, rb_b, ro_b, size):
        pltpu.sync_copy(src_hbm.at[pl.ds(cbase, size)], is_b)
        pltpu.sync_copy(dst_hbm.at[pl.ds(cbase, size)], id_b)
        pltpu.async_copy(ta_hbm.at[is_b], ra_b, sem).wait()
        pltpu.async_copy(tb_hbm.at[id_b], rb_b, sem).wait()
        compute(ra_b, rb_b, ro_b, size)
        pltpu.sync_copy(ro_b, out_hbm.at[pl.ds(cbase, size)])

    def chunk_loop(i, carry):
        do_chunk(base + i * CHUNK, is_v, id_v, ra_v, rb_v, ro_v, CHUNK)
        return carry
    lax.fori_loop(0, NFULL, chunk_loop, 0)
    do_chunk(base + NFULL * CHUNK, is_t, id_t, ra_t, rb_t, ro_t, TAIL)


def _edge_coeffs(ta, tb, src, dst):
    mesh = plsc.VectorSubcoreMesh(core_axis_name="c", subcore_axis_name="s")
    kern = functools.partial(
        pl.kernel,
        mesh=mesh,
        out_type=jax.ShapeDtypeStruct((E, NS), jnp.float32),
        scratch_types=[
            pltpu.VMEM((CHUNK,), jnp.int32),
            pltpu.VMEM((CHUNK,), jnp.int32),
            pltpu.VMEM((CHUNK, TW), jnp.float32),
            pltpu.VMEM((CHUNK, TW), jnp.float32),
            pltpu.VMEM((CHUNK, NS), jnp.float32),
            pltpu.VMEM((TAIL,), jnp.int32),
            pltpu.VMEM((TAIL,), jnp.int32),
            pltpu.VMEM((TAIL, TW), jnp.float32),
            pltpu.VMEM((TAIL, TW), jnp.float32),
            pltpu.VMEM((TAIL, NS), jnp.float32),
            pltpu.SemaphoreType.DMA,
        ],
    )(_edge_kernel)
    return kern(ta, tb, src, dst)


def _norm_body(agg_ref, rs_ref, wc_ref, o_ref):
    for h in range(HEADS):
        ah = agg_ref[:, h * D_IN:(h + 1) * D_IN]
        wh = wc_ref[:, h * D_OUT:(h + 1) * D_OUT]
        oh = jnp.dot(ah, wh, preferred_element_type=jnp.float32)
        o_ref[:, h * D_OUT:(h + 1) * D_OUT] = oh / rs_ref[:, h:h + 1]


def _project_normalize(agg, rowsum, wc):
    return pl.pallas_call(
        _norm_body,
        grid=(N // BN,),
        in_specs=[
            pl.BlockSpec((BN, HEADS * D_IN), lambda i: (i, 0)),
            pl.BlockSpec((BN, HEADS), lambda i: (i, 0)),
            pl.BlockSpec((D_IN, HEADS * D_OUT), lambda i: (0, 0)),
        ],
        out_specs=pl.BlockSpec((BN, HEADS * D_OUT), lambda i: (i, 0)),
        out_shape=jax.ShapeDtypeStruct((N, HEADS * D_OUT), jnp.float32),
    )(agg, rowsum, wc)


@jax.jit
def kernel(x, edge_index, W, att):
    src = edge_index[0, :]
    dst = edge_index[1, :]

    wc = jnp.transpose(W, (1, 0, 2)).reshape(D_IN, HEADS * D_OUT)
    v1 = jnp.einsum("hij,hj->ih", W, att[:, :D_OUT])
    v2 = jnp.einsum("hij,hj->ih", W, att[:, D_OUT:])
    v12 = jnp.zeros((D_IN, 2 * TW), jnp.float32)
    v12 = v12.at[:, :HEADS].set(v1).at[:, TW:TW + HEADS].set(v2)

    ss = _project(x, v12)
    ta = ss[:, :TW]
    tb = ss[:, TW:]

    ee16 = _edge_coeffs(ta, tb, src, dst)
    ee = ee16[:, :HEADS]

    e_rowsum = jax.ops.segment_sum(ee, src, num_segments=N)
    xg = x[dst, :]
    weighted = (ee[:, :, None] * xg[:, None, :]).reshape(E, HEADS * D_IN)
    agg = jax.ops.segment_sum(weighted, src, num_segments=N)

    return _project_normalize(agg, e_rowsum, wc)
